# baseline (speedup 1.0000x reference)
import functools

import jax
import jax.numpy as jnp
from jax import lax
from jax.experimental import pallas as pl
from jax.experimental.pallas import tpu as pltpu
from jax.experimental.pallas import tpu_sc as plsc

B = 16384
NCAT = 26
NNUM = 13
D = 32
CARD = 100000
NW = 32
CB = B // NW
L = 16
TS = CB + 1

_mesh = plsc.VectorSubcoreMesh(core_axis_name="c", subcore_axis_name="s")


@functools.partial(
    pl.kernel,
    out_type=jax.ShapeDtypeStruct((NNUM + NCAT, D, B), jnp.float32),
    mesh=_mesh,
    compiler_params=pltpu.CompilerParams(
        needs_layout_passes=False, use_tc_tiling_on_sc=False),
    scratch_types=[
        pltpu.VMEM((2, CB), jnp.int32),
        pltpu.VMEM((2, CB, D), jnp.float32),
        pltpu.VMEM((2, D, TS), jnp.float32),
        pltpu.VMEM((CB,), jnp.float32),
        pltpu.VMEM((NNUM * D * L,), jnp.float32),
        pltpu.VMEM((NNUM * D * L,), jnp.float32),
        pltpu.VMEM((NCAT * D,), jnp.float32),
        pltpu.SemaphoreType.DMA((2,)),
        pltpu.SemaphoreType.DMA((2,)),
    ],
)
def _emb(xnT, xcT, wrep, brep, table, cbf, out,
         idx_v, rows_v, tbuf, xn_v, wrep_v, brep_v, cb_v, gsem, osem):
    c = lax.axis_index("c")
    s = lax.axis_index("s")
    wid = s * 2 + c
    b0 = wid * CB

    pltpu.sync_copy(wrep, wrep_v)
    pltpu.sync_copy(brep, brep_v)
    pltpu.sync_copy(cbf, cb_v)

    lane = lax.broadcasted_iota(jnp.int32, (L,), 0)
    lane2 = lane + L

    def load_idx_and_gather(j, buf):
        pltpu.sync_copy(xcT.at[pl.ds(j * B + b0, CB)], idx_v.at[buf])
        off = jnp.full((L,), j * CARD, jnp.int32)

        @plsc.parallel_loop(0, CB // L, unroll=8)
        def off_p(p):
            idx_v[buf, pl.ds(p * L, L)] = idx_v[buf, pl.ds(p * L, L)] + off
        pltpu.async_copy(table.at[idx_v.at[buf]], rows_v.at[buf],
                         gsem.at[buf])

    def wait_gather(buf):
        pltpu.make_async_copy(table.at[idx_v.at[buf]], rows_v.at[buf],
                              gsem.at[buf]).wait()

    def issue_out(j, buf):
        pltpu.async_copy(tbuf.at[buf, :, pl.ds(0, CB)],
                         out.at[NNUM + j, :, pl.ds(b0, CB)],
                         osem.at[buf])

    def wait_out(j, buf):
        pltpu.make_async_copy(tbuf.at[buf, :, pl.ds(0, CB)],
                              out.at[NNUM + j, :, pl.ds(b0, CB)],
                              osem.at[buf]).wait()

    load_idx_and_gather(0, 0)
    load_idx_and_gather(1, 1)

    def num_i(i, carry):
        pltpu.sync_copy(xnT.at[pl.ds(i * B + b0, CB)], xn_v)

        def num_d(d, carry2):
            base = (i * D + d) * L
            wv = wrep_v[pl.ds(base, L)]
            bv = brep_v[pl.ds(base, L)]

            @plsc.parallel_loop(0, CB // L, unroll=8)
            def num_p(p):
                tbuf[0, d, pl.ds(p * L, L)] = xn_v[pl.ds(p * L, L)] * wv + bv

            return carry2

        lax.fori_loop(0, D, num_d, 0)
        pltpu.sync_copy(tbuf.at[0, :, pl.ds(0, CB)],
                        out.at[i, :, pl.ds(b0, CB)])
        return carry

    lax.fori_loop(0, NNUM, num_i, 0)

    def pair(k, carry):
        for buf in (0, 1):
            j = 2 * k + buf
            wait_gather(buf)

            @pl.when(k >= 1)
            def _(buf=buf, j=j):
                wait_out(j - 2, buf)

            cl = cb_v[pl.ds(j * D, L)]
            ch = cb_v[pl.ds(j * D + L, L)]

            @plsc.parallel_loop(0, CB, unroll=8)
            def trans_r(r, _buf=buf, _cl=cl, _ch=ch):
                rv = jnp.full((L,), r, jnp.int32)
                v0 = rows_v[_buf, r, pl.ds(0, L)] + _cl
                plsc.store_scatter(tbuf.at[_buf], [lane, rv], v0)
                v1 = rows_v[_buf, r, pl.ds(L, L)] + _ch
                plsc.store_scatter(tbuf.at[_buf], [lane2, rv], v1)

            @pl.when(k < (NCAT // 2) - 1)
            def _(buf=buf, j=j):
                load_idx_and_gather(j + 2, buf)

            issue_out(j, buf)
        return carry

    lax.fori_loop(0, NCAT // 2, pair, 0)

    wait_out(NCAT - 2, 0)
    wait_out(NCAT - 1, 1)


_V = 2600000
_CHUNK = 512


def _tpose_body(in_ref, out_ref):
    x = in_ref[...]
    out_ref[...] = (
        x.reshape(D, _CHUNK // 4, 4).transpose(1, 2, 0).reshape(_CHUNK // 4, 128)
    )


def _pack_table(table):
    grid = (_V + _CHUNK - 1) // _CHUNK
    packed = pl.pallas_call(
        _tpose_body,
        grid=(grid,),
        in_specs=[pl.BlockSpec((D, _CHUNK), lambda k: (0, k))],
        out_specs=pl.BlockSpec((_CHUNK // 4, 128), lambda k: (k, 0)),
        out_shape=jax.ShapeDtypeStruct((_V // 4, 128), jnp.float32),
    )(table.T)
    return packed.reshape(_V, D)


def kernel(x_num, x_cat, W_num, b_num, table, cat_bias):
    xnT = x_num.T.reshape(-1)
    xcT = x_cat.T.reshape(-1)
    wrep = jnp.repeat(W_num.reshape(-1)[:, None], L, axis=1).reshape(-1)
    brep = jnp.repeat(b_num.reshape(-1)[:, None], L, axis=1).reshape(-1)
    cbf = cat_bias.reshape(-1)
    t2 = _pack_table(table)
    outT = _emb(xnT, xcT, wrep, brep, t2, cbf)
    return jnp.transpose(outT, (2, 0, 1))


# speedup vs baseline: 5.2072x; 5.2072x over previous
import functools

import jax
import jax.numpy as jnp
from jax import lax
from jax.experimental import pallas as pl
from jax.experimental.pallas import tpu as pltpu
from jax.experimental.pallas import tpu_sc as plsc

B = 16384
NCAT = 26
NNUM = 13
D = 32
CARD = 100000
NW = 32
CB = B // NW
L = 16
TS = CB + 1

_mesh = plsc.VectorSubcoreMesh(core_axis_name="c", subcore_axis_name="s")


@functools.partial(
    pl.kernel,
    out_type=jax.ShapeDtypeStruct((NNUM + NCAT, D, B), jnp.float32),
    mesh=_mesh,
    compiler_params=pltpu.CompilerParams(
        needs_layout_passes=False, use_tc_tiling_on_sc=False),
    scratch_types=[
        pltpu.VMEM((2, CB), jnp.int32),
        pltpu.VMEM((2, CB, D), jnp.float32),
        pltpu.VMEM((2, D, TS), jnp.float32),
        pltpu.VMEM((NNUM * CB,), jnp.float32),
        pltpu.VMEM((NNUM * D * L,), jnp.float32),
        pltpu.VMEM((NNUM * D * L,), jnp.float32),
        pltpu.VMEM((NCAT * D,), jnp.float32),
        pltpu.SemaphoreType.DMA((2,)),
        pltpu.SemaphoreType.DMA((2,)),
        pltpu.SemaphoreType.DMA,
    ],
)
def _emb(xnT, xcT, wrep, brep, table, cbf, out,
         idx_v, rows_v, tbuf, xn_v, wrep_v, brep_v, cb_v, gsem, osem, xsem):
    c = lax.axis_index("c")
    s = lax.axis_index("s")
    wid = s * 2 + c
    b0 = wid * CB

    pltpu.sync_copy(wrep, wrep_v)
    pltpu.sync_copy(brep, brep_v)
    pltpu.sync_copy(cbf, cb_v)

    lane = lax.broadcasted_iota(jnp.int32, (L,), 0)
    lane2 = lane + L

    def load_idx_and_gather(j, buf):
        pltpu.sync_copy(xcT.at[pl.ds(j * B + b0, CB)], idx_v.at[buf])
        off = jnp.full((L,), j * CARD, jnp.int32)

        @plsc.parallel_loop(0, CB // L, unroll=8)
        def off_p(p):
            idx_v[buf, pl.ds(p * L, L)] = idx_v[buf, pl.ds(p * L, L)] + off
        pltpu.async_copy(table.at[idx_v.at[buf]], rows_v.at[buf],
                         gsem.at[buf])

    def wait_gather(buf):
        pltpu.make_async_copy(table.at[idx_v.at[buf]], rows_v.at[buf],
                              gsem.at[buf]).wait()

    def issue_out(j, buf):
        pltpu.async_copy(tbuf.at[buf, :, pl.ds(0, CB)],
                         out.at[NNUM + j, :, pl.ds(b0, CB)],
                         osem.at[buf])

    def wait_out(j, buf):
        pltpu.make_async_copy(tbuf.at[buf, :, pl.ds(0, CB)],
                              out.at[NNUM + j, :, pl.ds(b0, CB)],
                              osem.at[buf]).wait()

    def issue_num_out(i, buf):
        pltpu.async_copy(tbuf.at[buf, :, pl.ds(0, CB)],
                         out.at[i, :, pl.ds(b0, CB)],
                         osem.at[buf])

    def wait_num_out(i, buf):
        pltpu.make_async_copy(tbuf.at[buf, :, pl.ds(0, CB)],
                              out.at[i, :, pl.ds(b0, CB)],
                              osem.at[buf]).wait()

    load_idx_and_gather(0, 0)
    load_idx_and_gather(1, 1)
    for i in range(NNUM):
        pltpu.async_copy(xnT.at[pl.ds(i * B + b0, CB)],
                         xn_v.at[pl.ds(i * CB, CB)], xsem)

    for i in range(NNUM):
        buf = i % 2
        pltpu.make_async_copy(xnT.at[pl.ds(i * B + b0, CB)],
                              xn_v.at[pl.ds(i * CB, CB)], xsem).wait()
        if i >= 2:
            wait_num_out(i - 2, buf)

        def num_d(d, carry2, _i=i, _buf=buf):
            base = (_i * D + d) * L
            wv = wrep_v[pl.ds(base, L)]
            bv = brep_v[pl.ds(base, L)]

            @plsc.parallel_loop(0, CB // L, unroll=8)
            def num_p(p):
                tbuf[_buf, d, pl.ds(p * L, L)] = (
                    xn_v[pl.ds(_i * CB + p * L, L)] * wv + bv)

            return carry2

        lax.fori_loop(0, D, num_d, 0)
        issue_num_out(i, buf)

    wait_num_out(NNUM - 2, (NNUM - 2) % 2)
    wait_num_out(NNUM - 1, (NNUM - 1) % 2)

    def pair(k, carry):
        for buf in (0, 1):
            j = 2 * k + buf
            wait_gather(buf)

            @pl.when(k >= 1)
            def _(buf=buf, j=j):
                wait_out(j - 2, buf)

            cl = cb_v[pl.ds(j * D, L)]
            ch = cb_v[pl.ds(j * D + L, L)]

            @plsc.parallel_loop(0, CB, unroll=8)
            def trans_r(r, _buf=buf, _cl=cl, _ch=ch):
                rv = jnp.full((L,), r, jnp.int32)
                v0 = rows_v[_buf, r, pl.ds(0, L)] + _cl
                plsc.store_scatter(tbuf.at[_buf], [lane, rv], v0)
                v1 = rows_v[_buf, r, pl.ds(L, L)] + _ch
                plsc.store_scatter(tbuf.at[_buf], [lane2, rv], v1)

            @pl.when(k < (NCAT // 2) - 1)
            def _(buf=buf, j=j):
                load_idx_and_gather(j + 2, buf)

            issue_out(j, buf)
        return carry

    lax.fori_loop(0, NCAT // 2, pair, 0)

    wait_out(NCAT - 2, 0)
    wait_out(NCAT - 1, 1)


def kernel(x_num, x_cat, W_num, b_num, table, cat_bias):
    xnT = x_num.T.reshape(-1)
    xcT = x_cat.T.reshape(-1)
    wrep = jnp.repeat(W_num.reshape(-1)[:, None], L, axis=1).reshape(-1)
    brep = jnp.repeat(b_num.reshape(-1)[:, None], L, axis=1).reshape(-1)
    cbf = cat_bias.reshape(-1)
    outT = _emb(xnT, xcT, wrep, brep, table, cbf)
    return jnp.transpose(outT, (2, 0, 1))
